# Initial kernel scaffold; baseline (speedup 1.0000x reference)
import jax
import jax.numpy as jnp
from jax.experimental import pallas as pl


def kernel(x, edge_index, W0, b0, al0, ar0, bb0, W1, b1, al1, ar1, bb1, W2, b2, al2, ar2, bb2):
    raise NotImplementedError("write your pallas kernel here")


# speedup vs baseline: 35.6944x; 35.6944x over previous
import functools

import jax
import jax.numpy as jnp
from jax import lax
from jax.experimental import pallas as pl
from jax.experimental.pallas import tpu as pltpu
from jax.experimental.pallas import tpu_sc as plsc

N = 10000
E = 320000
D_IN = 128
HEADS = 8
PH = 16
HID = 128
NCLS = 40
SLOPE = 0.2

NC = 2
NS = 16
NW = NC * NS
LANES = 16

NP = 10240
BLK = 256
EW = E // NW
B = 80
NCHUNK = EW // B
RPT = NP // NS

_BIG = 1e30


def _leaky(v):
  return jnp.where(v >= 0, v, v * SLOPE)


def _prep_common(h_in, W_ref, b_ref, LR_ref, H_ref, T_ref, A_ref):
  h = jnp.dot(h_in, W_ref[...], preferred_element_type=jnp.float32)
  h = h + b_ref[...]
  H_ref[...] = h
  t = jnp.dot(h, LR_ref[...], preferred_element_type=jnp.float32)
  T_ref[...] = t
  blockmax = jnp.max(t[:, 0:8], axis=0, keepdims=True)
  cur = jnp.concatenate(
      [blockmax, jnp.full((1, 8), _BIG, jnp.float32)], axis=1)
  i = pl.program_id(0)

  @pl.when(i == 0)
  def _():
    A_ref[...] = cur

  @pl.when(i > 0)
  def _():
    A_ref[...] = jnp.maximum(A_ref[...], cur)


def _prep_first_body(x_ref, W_ref, b_ref, LR_ref, H_ref, T_ref, A_ref):
  _prep_common(x_ref[...], W_ref, b_ref, LR_ref, H_ref, T_ref, A_ref)


def _prep_mid_body(p_ref, bb_ref, W_ref, b_ref, LR_ref, H_ref, T_ref, A_ref):
  s = p_ref[0] + p_ref[1] + bb_ref[...]
  h_in = jnp.where(s > 0, s, jnp.exp(s) - 1.0)
  _prep_common(h_in, W_ref, b_ref, LR_ref, H_ref, T_ref, A_ref)


def _tc_prep_first(x_p, W, b_row, LR):
  grid = NP // BLK
  return pl.pallas_call(
      _prep_first_body,
      grid=(grid,),
      in_specs=[
          pl.BlockSpec((BLK, D_IN), lambda i: (i, 0)),
          pl.BlockSpec((D_IN, HEADS * PH), lambda i: (0, 0)),
          pl.BlockSpec((1, HEADS * PH), lambda i: (0, 0)),
          pl.BlockSpec((HEADS * PH, 16), lambda i: (0, 0)),
      ],
      out_specs=[
          pl.BlockSpec((BLK, HEADS * PH), lambda i: (i, 0)),
          pl.BlockSpec((BLK, 16), lambda i: (i, 0)),
          pl.BlockSpec((1, 16), lambda i: (0, 0)),
      ],
      out_shape=[
          jax.ShapeDtypeStruct((NP, HEADS * PH), jnp.float32),
          jax.ShapeDtypeStruct((NP, 16), jnp.float32),
          jax.ShapeDtypeStruct((1, 16), jnp.float32),
      ],
  )(x_p, W, b_row, LR)


def _tc_prep_mid(p, bb_row, W, b_row, LR, d_in, d_out):
  grid = NP // BLK
  return pl.pallas_call(
      _prep_mid_body,
      grid=(grid,),
      in_specs=[
          pl.BlockSpec((2, BLK, d_in), lambda i: (0, i, 0)),
          pl.BlockSpec((1, d_in), lambda i: (0, 0)),
          pl.BlockSpec((d_in, d_out), lambda i: (0, 0)),
          pl.BlockSpec((1, d_out), lambda i: (0, 0)),
          pl.BlockSpec((d_out, 16), lambda i: (0, 0)),
      ],
      out_specs=[
          pl.BlockSpec((BLK, d_out), lambda i: (i, 0)),
          pl.BlockSpec((BLK, 16), lambda i: (i, 0)),
          pl.BlockSpec((1, 16), lambda i: (0, 0)),
      ],
      out_shape=[
          jax.ShapeDtypeStruct((NP, d_out), jnp.float32),
          jax.ShapeDtypeStruct((NP, 16), jnp.float32),
          jax.ShapeDtypeStruct((1, 16), jnp.float32),
      ],
  )(p, bb_row, W, b_row, LR)


def _comb_s_body(sp_ref, sinv_ref):
  sinv_ref[...] = 1.0 / (sp_ref[0] + sp_ref[1] + 1e-16)


def _tc_comb_s(sp):
  grid = NP // BLK
  return pl.pallas_call(
      _comb_s_body,
      grid=(grid,),
      in_specs=[pl.BlockSpec((2, BLK, 16), lambda i: (0, i, 0))],
      out_specs=pl.BlockSpec((BLK, 16), lambda i: (i, 0)),
      out_shape=jax.ShapeDtypeStruct((NP, 16), jnp.float32),
  )(sp)


def _final_body(p_ref, bb_ref, o_ref):
  o_ref[...] = p_ref[0] + p_ref[1] + bb_ref[...]


def _tc_final(p, bb_row, d_out):
  grid = NP // BLK
  return pl.pallas_call(
      _final_body,
      grid=(grid,),
      in_specs=[
          pl.BlockSpec((2, BLK, d_out), lambda i: (0, i, 0)),
          pl.BlockSpec((1, d_out), lambda i: (0, 0)),
      ],
      out_specs=pl.BlockSpec((BLK, d_out), lambda i: (i, 0)),
      out_shape=jax.ShapeDtypeStruct((NP, d_out), jnp.float32),
  )(p, bb_row)


_MESH = plsc.VectorSubcoreMesh(core_axis_name="c", subcore_axis_name="s")


def _edge_w(ts, td, av):
  rot_idx = (lax.iota(jnp.int32, LANES) & 7) + 8
  rot = jnp.take_along_axis(td, rot_idx, axis=0)
  e = _leaky(ts + rot)
  m = _leaky(rot + av)
  return jnp.exp(e - m)


def _passA_body(esrc, edst, T, avec, s_out, src_v, dst_v, tsrc, tdst, wbuf,
                a_v, s_sh, sem1, sem2):
  cid = lax.axis_index("c")
  sid = lax.axis_index("s")
  wid = cid * NS + sid

  def zero_row(i, c):
    wbuf[i, :] = jnp.zeros((LANES,), jnp.float32)
    return c

  lax.fori_loop(0, B, zero_row, 0)

  def zero_sh(k, c):
    pltpu.sync_copy(wbuf, s_sh.at[pl.ds(sid * RPT + k * B, B)])
    return c

  lax.fori_loop(0, RPT // B, zero_sh, 0)
  plsc.subcore_barrier()

  pltpu.sync_copy(avec, a_v)
  av = a_v[:]
  base_w = wid * EW

  def chunk(ci, c):
    base = base_w + ci * B
    pltpu.sync_copy(esrc.at[pl.ds(base, B)], src_v)
    pltpu.sync_copy(edst.at[pl.ds(base, B)], dst_v)
    d1 = pltpu.async_copy(T.at[src_v], tsrc, sem1)
    d2 = pltpu.async_copy(T.at[dst_v], tdst, sem2)
    d1.wait()
    d2.wait()

    def per_edge(i, cc):
      wbuf[i, :] = _edge_w(tsrc[i, :], tdst[i, :], av)
      return cc

    lax.fori_loop(0, B, per_edge, 0)
    pltpu.sync_copy(wbuf, s_sh.at[dst_v], add=True)
    return c

  lax.fori_loop(0, NCHUNK, chunk, 0)
  plsc.subcore_barrier()
  pltpu.sync_copy(s_sh.at[pl.ds(sid * RPT, RPT)],
                  s_out.at[cid, pl.ds(sid * RPT, RPT)])


_sc_passA = functools.partial(
    pl.kernel,
    out_type=jax.ShapeDtypeStruct((NC, NP, 16), jnp.float32),
    mesh=_MESH,
    compiler_params=pltpu.CompilerParams(use_tc_tiling_on_sc=False),
    scratch_types=[
        pltpu.VMEM((B,), jnp.int32),
        pltpu.VMEM((B,), jnp.int32),
        pltpu.VMEM((B, 16), jnp.float32),
        pltpu.VMEM((B, 16), jnp.float32),
        pltpu.VMEM((B, 16), jnp.float32),
        pltpu.VMEM((LANES,), jnp.float32),
        pltpu.VMEM_SHARED((NP, 16), jnp.float32),
        pltpu.SemaphoreType.DMA,
        pltpu.SemaphoreType.DMA,
    ],
)(_passA_body)


def _make_passB(d_row, n_heads):
  n_chunks = d_row // LANES
  splat_head = [(c if n_heads == HEADS else 0) for c in range(n_chunks)]

  def body(esrc, edst, T, avec, sinv, htab, out, src_v, dst_v, tsrc, tdst,
           sv, hrows, a_v, o_sh, sem1, sem2, sem3, sem4):
    cid = lax.axis_index("c")
    sid = lax.axis_index("s")
    wid = cid * NS + sid

    def zero_row(i, c):
      for j in range(n_chunks):
        hrows[i, pl.ds(j * LANES, LANES)] = jnp.zeros((LANES,), jnp.float32)
      return c

    lax.fori_loop(0, B, zero_row, 0)

    def zero_sh(k, c):
      pltpu.sync_copy(hrows, o_sh.at[pl.ds(sid * RPT + k * B, B)])
      return c

    lax.fori_loop(0, RPT // B, zero_sh, 0)
    plsc.subcore_barrier()

    pltpu.sync_copy(avec, a_v)
    av = a_v[:]
    base_w = wid * EW

    def chunk(ci, c):
      base = base_w + ci * B
      pltpu.sync_copy(esrc.at[pl.ds(base, B)], src_v)
      pltpu.sync_copy(edst.at[pl.ds(base, B)], dst_v)
      d1 = pltpu.async_copy(T.at[src_v], tsrc, sem1)
      d2 = pltpu.async_copy(T.at[dst_v], tdst, sem2)
      d3 = pltpu.async_copy(sinv.at[dst_v], sv, sem3)
      d4 = pltpu.async_copy(htab.at[src_v], hrows, sem4)
      d1.wait()
      d2.wait()
      d3.wait()
      d4.wait()

      def per_edge(i, cc):
        w = _edge_w(tsrc[i, :], tdst[i, :], av)
        attn = w * sv[i, :]
        for j in range(n_chunks):
          sp = jnp.take_along_axis(
              attn, jnp.full((LANES,), splat_head[j], jnp.int32), axis=0)
          hrows[i, pl.ds(j * LANES, LANES)] = (
              hrows[i, pl.ds(j * LANES, LANES)] * sp)
        return cc

      lax.fori_loop(0, B, per_edge, 0)
      pltpu.sync_copy(hrows, o_sh.at[dst_v], add=True)
      return c

    lax.fori_loop(0, NCHUNK, chunk, 0)
    plsc.subcore_barrier()
    pltpu.sync_copy(o_sh.at[pl.ds(sid * RPT, RPT)],
                    out.at[cid, pl.ds(sid * RPT, RPT)])

  return functools.partial(
      pl.kernel,
      out_type=jax.ShapeDtypeStruct((NC, NP, d_row), jnp.float32),
      mesh=_MESH,
      compiler_params=pltpu.CompilerParams(use_tc_tiling_on_sc=False),
      scratch_types=[
          pltpu.VMEM((B,), jnp.int32),
          pltpu.VMEM((B,), jnp.int32),
          pltpu.VMEM((B, 16), jnp.float32),
          pltpu.VMEM((B, 16), jnp.float32),
          pltpu.VMEM((B, 16), jnp.float32),
          pltpu.VMEM((B, d_row), jnp.float32),
          pltpu.VMEM((LANES,), jnp.float32),
          pltpu.VMEM_SHARED((NP, d_row), jnp.float32),
          pltpu.SemaphoreType.DMA,
          pltpu.SemaphoreType.DMA,
          pltpu.SemaphoreType.DMA,
          pltpu.SemaphoreType.DMA,
      ],
  )(body)


_sc_passB_128 = _make_passB(128, HEADS)
_sc_passB_48 = _make_passB(48, 1)


def _build_lr(al, ar, d, heads, ph):
  rows = jnp.arange(d)
  hcol = rows // ph
  lr = jnp.zeros((d, 16), jnp.float32)
  lr = lr.at[rows, hcol].set(al.reshape(-1))
  lr = lr.at[rows, hcol + 8].set(ar.reshape(-1))
  return lr


def kernel(x, edge_index, W0, b0, al0, ar0, bb0, W1, b1, al1, ar1, bb1,
           W2, b2, al2, ar2, bb2):
  ei = edge_index.astype(jnp.int32)
  esrc = ei[0]
  edst = ei[1]
  x_p = jnp.pad(x, ((0, NP - N), (0, 0)))

  LR0 = _build_lr(al0, ar0, 128, HEADS, PH)
  LR1 = _build_lr(al1, ar1, 128, HEADS, PH)
  al2p = jnp.pad(al2.reshape(-1), (0, 8))
  ar2p = jnp.pad(ar2.reshape(-1), (0, 8))
  LR2 = _build_lr(al2p, ar2p, 48, 1, 48)
  W2p = jnp.pad(W2, ((0, 0), (0, 8)))
  b2p = jnp.pad(b2, (0, 8))
  bb2p = jnp.pad(bb2, (0, 8))

  H0, T0, A0 = _tc_prep_first(x_p, W0, b0.reshape(1, -1), LR0)
  A0v = A0.reshape(16)
  sp0 = _sc_passA(esrc, edst, T0, A0v)
  Sinv0 = _tc_comb_s(sp0)
  op0 = _sc_passB_128(esrc, edst, T0, A0v, Sinv0, H0)

  H1, T1, A1 = _tc_prep_mid(op0, bb0.reshape(1, -1), W1, b1.reshape(1, -1),
                            LR1, 128, 128)
  A1v = A1.reshape(16)
  sp1 = _sc_passA(esrc, edst, T1, A1v)
  Sinv1 = _tc_comb_s(sp1)
  op1 = _sc_passB_128(esrc, edst, T1, A1v, Sinv1, H1)

  H2, T2, A2 = _tc_prep_mid(op1, bb1.reshape(1, -1), W2p, b2p.reshape(1, -1),
                            LR2, 128, 48)
  A2v = A2.reshape(16)
  sp2 = _sc_passA(esrc, edst, T2, A2v)
  Sinv2 = _tc_comb_s(sp2)
  op2 = _sc_passB_48(esrc, edst, T2, A2v, Sinv2, H2)

  outf = _tc_final(op2, bb2p.reshape(1, -1), 48)
  return outf[:N, :NCLS]


# speedup vs baseline: 37.6181x; 1.0539x over previous
import functools

import jax
import jax.numpy as jnp
from jax import lax
from jax.experimental import pallas as pl
from jax.experimental.pallas import tpu as pltpu
from jax.experimental.pallas import tpu_sc as plsc

N = 10000
E = 320000
D_IN = 128
HEADS = 8
PH = 16
HID = 128
NCLS = 40
SLOPE = 0.2

NC = 2
NS = 16
NW = NC * NS
LANES = 16

NP = 10240
BLK = 256
EW = E // NW
B = 40
NCHUNK = EW // B
RPT = NP // NS

_BIG = 1e30


def _leaky(v):
  return jnp.where(v >= 0, v, v * SLOPE)


def _prep_common(h_in, W_ref, b_ref, LR_ref, HA_ref, T_ref, A_ref, d_out):
  h = jnp.dot(h_in, W_ref[...], preferred_element_type=jnp.float32)
  h = h + b_ref[...]
  t = jnp.dot(h, LR_ref[...], preferred_element_type=jnp.float32)
  T_ref[...] = t
  HA_ref[:, 0:d_out] = h
  HA_ref[:, d_out:d_out + 8] = t[:, 0:8]
  HA_ref[:, d_out + 8:d_out + 16] = jnp.zeros((h.shape[0], 8), jnp.float32)
  blockmax = jnp.max(t[:, 0:8], axis=0, keepdims=True)
  cur = jnp.concatenate(
      [blockmax, jnp.full((1, 8), _BIG, jnp.float32)], axis=1)
  i = pl.program_id(0)

  @pl.when(i == 0)
  def _():
    A_ref[...] = cur

  @pl.when(i > 0)
  def _():
    A_ref[...] = jnp.maximum(A_ref[...], cur)


def _tc_prep(p, bb_row, W, b_row, LR, d_in, d_out, first):
  grid = NP // BLK

  def body(p_ref, bb_ref, W_ref, b_ref, LR_ref, HA_ref, T_ref, A_ref):
    if first:
      h_in = p_ref[...]
    else:
      s = p_ref[0] + p_ref[1] + bb_ref[...]
      h_in = jnp.where(s > 0, s, jnp.exp(s) - 1.0)
    _prep_common(h_in, W_ref, b_ref, LR_ref, HA_ref, T_ref, A_ref, d_out)

  in_spec_p = (
      pl.BlockSpec((BLK, d_in), lambda i: (i, 0)) if first
      else pl.BlockSpec((2, BLK, d_in), lambda i: (0, i, 0)))
  return pl.pallas_call(
      body,
      grid=(grid,),
      in_specs=[
          in_spec_p,
          pl.BlockSpec((1, d_in), lambda i: (0, 0)),
          pl.BlockSpec((d_in, d_out), lambda i: (0, 0)),
          pl.BlockSpec((1, d_out), lambda i: (0, 0)),
          pl.BlockSpec((d_out, 16), lambda i: (0, 0)),
      ],
      out_specs=[
          pl.BlockSpec((BLK, d_out + 16), lambda i: (i, 0)),
          pl.BlockSpec((BLK, 16), lambda i: (i, 0)),
          pl.BlockSpec((1, 16), lambda i: (0, 0)),
      ],
      out_shape=[
          jax.ShapeDtypeStruct((NP, d_out + 16), jnp.float32),
          jax.ShapeDtypeStruct((NP, 16), jnp.float32),
          jax.ShapeDtypeStruct((1, 16), jnp.float32),
      ],
  )(p, bb_row, W, b_row, LR)


def _comb_ts_body(sp_ref, t_ref, ts_ref):
  sinv = 1.0 / (sp_ref[0, :, 0:8] + sp_ref[1, :, 0:8] + 1e-16)
  ts_ref[...] = jnp.concatenate([sinv, t_ref[:, 8:16]], axis=1)


def _tc_comb_ts(sp, t):
  grid = NP // BLK
  return pl.pallas_call(
      _comb_ts_body,
      grid=(grid,),
      in_specs=[
          pl.BlockSpec((2, BLK, 16), lambda i: (0, i, 0)),
          pl.BlockSpec((BLK, 16), lambda i: (i, 0)),
      ],
      out_specs=pl.BlockSpec((BLK, 16), lambda i: (i, 0)),
      out_shape=jax.ShapeDtypeStruct((NP, 16), jnp.float32),
  )(sp, t)


def _final_body(p_ref, bb_ref, o_ref):
  o_ref[...] = p_ref[0] + p_ref[1] + bb_ref[...]


def _tc_final(p, bb_row, d_out):
  grid = NP // BLK
  return pl.pallas_call(
      _final_body,
      grid=(grid,),
      in_specs=[
          pl.BlockSpec((2, BLK, d_out), lambda i: (0, i, 0)),
          pl.BlockSpec((1, d_out), lambda i: (0, 0)),
      ],
      out_specs=pl.BlockSpec((BLK, d_out), lambda i: (i, 0)),
      out_shape=jax.ShapeDtypeStruct((NP, d_out), jnp.float32),
  )(p, bb_row)


_MESH = plsc.VectorSubcoreMesh(core_axis_name="c", subcore_axis_name="s")


def _edge_w(ts, td, av):
  rot_idx = (lax.iota(jnp.int32, LANES) & 7) + 8
  rot = jnp.take_along_axis(td, rot_idx, axis=0)
  e = _leaky(ts + rot)
  m = _leaky(rot + av)
  return jnp.exp(e - m)


def _passA_body(esrc, edst, T, avec, s_out, src_all, dst_all, tsrc, tdst,
                wbuf, a_v, s_sh, semA, semB):
  cid = lax.axis_index("c")
  sid = lax.axis_index("s")
  wid = cid * NS + sid
  tsrcs = [tsrc.at[0], tsrc.at[1]]
  tdsts = [tdst.at[0], tdst.at[1]]
  wbufs = [wbuf.at[0], wbuf.at[1]]
  sems = [semA, semB]

  def zero_row(i, c):
    wbuf[0, i, :] = jnp.zeros((LANES,), jnp.float32)
    return c

  lax.fori_loop(0, B, zero_row, 0)

  def zero_sh(k, c):
    pltpu.sync_copy(wbufs[0], s_sh.at[pl.ds(sid * RPT + k * B, B)])
    return c

  lax.fori_loop(0, RPT // B, zero_sh, 0)
  plsc.subcore_barrier()

  pltpu.sync_copy(avec, a_v)
  av = a_v[:]
  pltpu.sync_copy(esrc.at[wid], src_all)
  pltpu.sync_copy(edst.at[wid], dst_all)

  def fire(ci, b):
    pltpu.async_copy(T.at[src_all.at[ci]], tsrcs[b], sems[b])
    pltpu.async_copy(T.at[dst_all.at[ci]], tdsts[b], sems[b])

  def consume(ci, b):
    pltpu.make_async_copy(T.at[src_all.at[ci]], tsrcs[b], sems[b]).wait()
    pltpu.make_async_copy(T.at[dst_all.at[ci]], tdsts[b], sems[b]).wait()

    def per_edge(i, cc):
      wbuf[b, i, :] = _edge_w(tsrc[b, i, :], tdst[b, i, :], av)
      return cc

    lax.fori_loop(0, B, per_edge, 0)
    pltpu.sync_copy(wbufs[b], s_sh.at[dst_all.at[ci]], add=True)

  fire(0, 0)

  def pair(k, c):
    ci = 2 * k
    fire(ci + 1, 1)
    consume(ci, 0)
    fire(ci + 2, 0)
    consume(ci + 1, 1)
    return c

  lax.fori_loop(0, NCHUNK // 2 - 1, pair, 0)
  fire(NCHUNK - 1, 1)
  consume(NCHUNK - 2, 0)
  consume(NCHUNK - 1, 1)

  plsc.subcore_barrier()
  pltpu.sync_copy(s_sh.at[pl.ds(sid * RPT, RPT)],
                  s_out.at[cid, pl.ds(sid * RPT, RPT)])


_sc_passA = functools.partial(
    pl.kernel,
    out_type=jax.ShapeDtypeStruct((NC, NP, 16), jnp.float32),
    mesh=_MESH,
    compiler_params=pltpu.CompilerParams(use_tc_tiling_on_sc=False),
    scratch_types=[
        pltpu.VMEM((NCHUNK, B), jnp.int32),
        pltpu.VMEM((NCHUNK, B), jnp.int32),
        pltpu.VMEM((2, B, 16), jnp.float32),
        pltpu.VMEM((2, B, 16), jnp.float32),
        pltpu.VMEM((2, B, 16), jnp.float32),
        pltpu.VMEM((LANES,), jnp.float32),
        pltpu.VMEM_SHARED((NP, 16), jnp.float32),
        pltpu.SemaphoreType.DMA,
        pltpu.SemaphoreType.DMA,
    ],
)(_passA_body)


def _make_passB(d_row, n_heads):
  n_chunks = d_row // LANES
  splat_head = [(c if n_heads == HEADS else 0) for c in range(n_chunks)]
  ha_w = d_row + 16

  def body(esrc, edst, HA, avec, TS, out, src_all, dst_all, ha, ts, msg,
           a_v, o_sh, semA, semB):
    cid = lax.axis_index("c")
    sid = lax.axis_index("s")
    wid = cid * NS + sid
    has = [ha.at[0], ha.at[1]]
    tss = [ts.at[0], ts.at[1]]
    msgs = [msg.at[0], msg.at[1]]
    sems = [semA, semB]

    def zero_row(i, c):
      for j in range(n_chunks):
        msg[0, i, pl.ds(j * LANES, LANES)] = jnp.zeros((LANES,), jnp.float32)
      return c

    lax.fori_loop(0, B, zero_row, 0)

    def zero_sh(k, c):
      pltpu.sync_copy(msgs[0], o_sh.at[pl.ds(sid * RPT + k * B, B)])
      return c

    lax.fori_loop(0, RPT // B, zero_sh, 0)
    plsc.subcore_barrier()

    pltpu.sync_copy(avec, a_v)
    av = a_v[:]
    pltpu.sync_copy(esrc.at[wid], src_all)
    pltpu.sync_copy(edst.at[wid], dst_all)

    def fire(ci, b):
      pltpu.async_copy(HA.at[src_all.at[ci]], has[b], sems[b])
      pltpu.async_copy(TS.at[dst_all.at[ci]], tss[b], sems[b])

    def consume(ci, b):
      pltpu.make_async_copy(HA.at[src_all.at[ci]], has[b], sems[b]).wait()
      pltpu.make_async_copy(TS.at[dst_all.at[ci]], tss[b], sems[b]).wait()

      def per_edge(i, cc):
        tsr = ha[b, i, pl.ds(d_row, LANES)]
        tdr = ts[b, i, :]
        w = _edge_w(tsr, tdr, av)
        attn = w * tdr
        for j in range(n_chunks):
          sp = jnp.take_along_axis(
              attn, jnp.full((LANES,), splat_head[j], jnp.int32), axis=0)
          msg[b, i, pl.ds(j * LANES, LANES)] = (
              ha[b, i, pl.ds(j * LANES, LANES)] * sp)
        return cc

      lax.fori_loop(0, B, per_edge, 0)
      pltpu.sync_copy(msgs[b], o_sh.at[dst_all.at[ci]], add=True)

    fire(0, 0)

    def pair(k, c):
      ci = 2 * k
      fire(ci + 1, 1)
      consume(ci, 0)
      fire(ci + 2, 0)
      consume(ci + 1, 1)
      return c

    lax.fori_loop(0, NCHUNK // 2 - 1, pair, 0)
    fire(NCHUNK - 1, 1)
    consume(NCHUNK - 2, 0)
    consume(NCHUNK - 1, 1)

    plsc.subcore_barrier()
    pltpu.sync_copy(o_sh.at[pl.ds(sid * RPT, RPT)],
                    out.at[cid, pl.ds(sid * RPT, RPT)])

  return functools.partial(
      pl.kernel,
      out_type=jax.ShapeDtypeStruct((NC, NP, d_row), jnp.float32),
      mesh=_MESH,
      compiler_params=pltpu.CompilerParams(use_tc_tiling_on_sc=False),
      scratch_types=[
          pltpu.VMEM((NCHUNK, B), jnp.int32),
          pltpu.VMEM((NCHUNK, B), jnp.int32),
          pltpu.VMEM((2, B, ha_w), jnp.float32),
          pltpu.VMEM((2, B, 16), jnp.float32),
          pltpu.VMEM((2, B, d_row), jnp.float32),
          pltpu.VMEM((LANES,), jnp.float32),
          pltpu.VMEM_SHARED((NP, d_row), jnp.float32),
          pltpu.SemaphoreType.DMA,
          pltpu.SemaphoreType.DMA,
      ],
  )(body)


_sc_passB_128 = _make_passB(128, HEADS)


def _build_lr(al, ar, d, ph):
  rows = jnp.arange(d)
  hcol = rows // ph
  lr = jnp.zeros((d, 16), jnp.float32)
  lr = lr.at[rows, hcol].set(al.reshape(-1))
  lr = lr.at[rows, hcol + 8].set(ar.reshape(-1))
  return lr


def _layer(esrc, edst, p, bb_row, W, b_row, LR, d_in, d_out, first, passB):
  HA, T, A = _tc_prep(p, bb_row, W, b_row, LR, d_in, d_out, first)
  Av = A.reshape(16)
  sp = _sc_passA(esrc, edst, T, Av)
  TS = _tc_comb_ts(sp, T)
  return passB(esrc, edst, HA, Av, TS)


def kernel(x, edge_index, W0, b0, al0, ar0, bb0, W1, b1, al1, ar1, bb1,
           W2, b2, al2, ar2, bb2):
  ei = edge_index.astype(jnp.int32)
  esrc = ei[0].reshape(NW, NCHUNK, B)
  edst = ei[1].reshape(NW, NCHUNK, B)
  x_p = jnp.pad(x, ((0, NP - N), (0, 0)))

  LR0 = _build_lr(al0, ar0, 128, PH)
  LR1 = _build_lr(al1, ar1, 128, PH)
  rows2 = jnp.arange(NCLS)
  LR2 = jnp.zeros((128, 16), jnp.float32)
  for _j in range(8):
    LR2 = LR2.at[rows2, _j].set(al2.reshape(-1))
    LR2 = LR2.at[rows2, 8 + _j].set(ar2.reshape(-1))
  W2p = jnp.pad(W2, ((0, 0), (0, 128 - NCLS)))
  b2p = jnp.pad(b2, (0, 128 - NCLS))
  bb2p = jnp.pad(bb2, (0, 128 - NCLS))

  op0 = _layer(esrc, edst, x_p, b0.reshape(1, -1), W0, b0.reshape(1, -1),
               LR0, 128, 128, True, _sc_passB_128)
  op1 = _layer(esrc, edst, op0, bb0.reshape(1, -1), W1, b1.reshape(1, -1),
               LR1, 128, 128, False, _sc_passB_128)
  op2 = _layer(esrc, edst, op1, bb1.reshape(1, -1), W2p, b2p.reshape(1, -1),
               LR2, 128, 128, False, _sc_passB_128)

  outf = _tc_final(op2, bb2p.reshape(1, -1), 128)
  return outf[:N, :NCLS]


# speedup vs baseline: 41.1814x; 1.0947x over previous
import functools

import jax
import jax.numpy as jnp
from jax import lax
from jax.experimental import pallas as pl
from jax.experimental.pallas import tpu as pltpu
from jax.experimental.pallas import tpu_sc as plsc

N = 10000
E = 320000
D_IN = 128
HEADS = 8
PH = 16
HID = 128
NCLS = 40
SLOPE = 0.2

NC = 2
NS = 16
NW = NC * NS
LANES = 16

NP = 10240
BLK = 256
EW = E // NW
B = 40
NCHUNK = EW // B
RPT = NP // NS

_BIG = 1e30


def _leaky(v):
  return jnp.where(v >= 0, v, v * SLOPE)


def _prep_common(h_in, W_ref, b_ref, LR_ref, HA_ref, T_ref, A_ref, d_out):
  h = jnp.dot(h_in, W_ref[...], preferred_element_type=jnp.float32)
  h = h + b_ref[...]
  t = jnp.dot(h, LR_ref[...], preferred_element_type=jnp.float32)
  T_ref[...] = t
  HA_ref[:, 0:d_out] = h
  HA_ref[:, d_out:d_out + 8] = t[:, 0:8]
  HA_ref[:, d_out + 8:d_out + 16] = jnp.zeros((h.shape[0], 8), jnp.float32)
  blockmax = jnp.max(t[:, 0:8], axis=0, keepdims=True)
  cur = jnp.concatenate(
      [blockmax, jnp.full((1, 8), _BIG, jnp.float32)], axis=1)
  i = pl.program_id(0)

  @pl.when(i == 0)
  def _():
    A_ref[...] = cur

  @pl.when(i > 0)
  def _():
    A_ref[...] = jnp.maximum(A_ref[...], cur)


def _tc_prep(p, bb_row, W, b_row, LR, d_in, d_out, first):
  grid = NP // BLK

  def body(p_ref, bb_ref, W_ref, b_ref, LR_ref, HA_ref, T_ref, A_ref):
    if first:
      h_in = p_ref[...]
    else:
      s = p_ref[0] + p_ref[1] + bb_ref[...]
      h_in = jnp.where(s > 0, s, jnp.exp(s) - 1.0)
    _prep_common(h_in, W_ref, b_ref, LR_ref, HA_ref, T_ref, A_ref, d_out)

  in_spec_p = (
      pl.BlockSpec((BLK, d_in), lambda i: (i, 0)) if first
      else pl.BlockSpec((2, BLK, d_in), lambda i: (0, i, 0)))
  return pl.pallas_call(
      body,
      grid=(grid,),
      in_specs=[
          in_spec_p,
          pl.BlockSpec((1, d_in), lambda i: (0, 0)),
          pl.BlockSpec((d_in, d_out), lambda i: (0, 0)),
          pl.BlockSpec((1, d_out), lambda i: (0, 0)),
          pl.BlockSpec((d_out, 16), lambda i: (0, 0)),
      ],
      out_specs=[
          pl.BlockSpec((BLK, d_out + 16), lambda i: (i, 0)),
          pl.BlockSpec((BLK, 16), lambda i: (i, 0)),
          pl.BlockSpec((1, 16), lambda i: (0, 0)),
      ],
      out_shape=[
          jax.ShapeDtypeStruct((NP, d_out + 16), jnp.float32),
          jax.ShapeDtypeStruct((NP, 16), jnp.float32),
          jax.ShapeDtypeStruct((1, 16), jnp.float32),
      ],
  )(p, bb_row, W, b_row, LR)


def _comb_ts_body(sp_ref, t_ref, ts_ref):
  sinv = 1.0 / (sp_ref[0, :, 0:8] + sp_ref[1, :, 0:8] + 1e-16)
  ts_ref[...] = jnp.concatenate([sinv, t_ref[:, 8:16]], axis=1)


def _tc_comb_ts(sp, t):
  grid = NP // BLK
  return pl.pallas_call(
      _comb_ts_body,
      grid=(grid,),
      in_specs=[
          pl.BlockSpec((2, BLK, 16), lambda i: (0, i, 0)),
          pl.BlockSpec((BLK, 16), lambda i: (i, 0)),
      ],
      out_specs=pl.BlockSpec((BLK, 16), lambda i: (i, 0)),
      out_shape=jax.ShapeDtypeStruct((NP, 16), jnp.float32),
  )(sp, t)


def _final_body(p_ref, bb_ref, o_ref):
  o_ref[...] = p_ref[0] + p_ref[1] + bb_ref[...]


def _tc_final(p, bb_row, d_out):
  grid = NP // BLK
  return pl.pallas_call(
      _final_body,
      grid=(grid,),
      in_specs=[
          pl.BlockSpec((2, BLK, d_out), lambda i: (0, i, 0)),
          pl.BlockSpec((1, d_out), lambda i: (0, 0)),
      ],
      out_specs=pl.BlockSpec((BLK, d_out), lambda i: (i, 0)),
      out_shape=jax.ShapeDtypeStruct((NP, d_out), jnp.float32),
  )(p, bb_row)


_MESH = plsc.VectorSubcoreMesh(core_axis_name="c", subcore_axis_name="s")


def _edge_w(ts, td, av):
  rot_idx = (lax.iota(jnp.int32, LANES) & 7) + 8
  rot = jnp.take_along_axis(td, rot_idx, axis=0)
  e = _leaky(ts + rot)
  m = _leaky(rot + av)
  return jnp.exp(e - m)


def _passA_body(esrc, edst, T, avec, s_out, src_all, dst_all, tsrc, tdst,
                wbuf, a_v, s_sh, semA, semB, semSA, semSB):
  cid = lax.axis_index("c")
  sid = lax.axis_index("s")
  wid = cid * NS + sid
  tsrcs = [tsrc.at[0], tsrc.at[1]]
  tdsts = [tdst.at[0], tdst.at[1]]
  wbufs = [wbuf.at[0], wbuf.at[1]]
  sems = [semA, semB]
  ssems = [semSA, semSB]

  def zero_row(i, c):
    wbuf[0, i, :] = jnp.zeros((LANES,), jnp.float32)
    return c

  lax.fori_loop(0, B, zero_row, 0)

  def zero_sh(k, c):
    pltpu.sync_copy(wbufs[0], s_sh.at[pl.ds(sid * RPT + k * B, B)])
    return c

  lax.fori_loop(0, RPT // B, zero_sh, 0)
  plsc.subcore_barrier()

  pltpu.sync_copy(avec, a_v)
  av = a_v[:]
  pltpu.sync_copy(esrc.at[wid], src_all)
  pltpu.sync_copy(edst.at[wid], dst_all)

  def fire(ci, b):
    pltpu.async_copy(T.at[src_all.at[ci]], tsrcs[b], sems[b])
    pltpu.async_copy(T.at[dst_all.at[ci]], tdsts[b], sems[b])

  def wait_gather(ci, b):
    pltpu.make_async_copy(T.at[src_all.at[ci]], tsrcs[b], sems[b]).wait()
    pltpu.make_async_copy(T.at[dst_all.at[ci]], tdsts[b], sems[b]).wait()

  def compute(ci, b):
    def per_edge(i, cc):
      wbuf[b, i, :] = _edge_w(tsrc[b, i, :], tdst[b, i, :], av)
      return cc

    lax.fori_loop(0, B, per_edge, 0)

  def fire_sc(ci, b):
    pltpu.async_copy(wbufs[b], s_sh.at[dst_all.at[ci]], ssems[b], add=True)

  def wait_sc(ci, b):
    pltpu.make_async_copy(wbufs[b], s_sh.at[dst_all.at[ci]], ssems[b]).wait()

  fire(0, 0)
  fire(1, 1)
  wait_gather(0, 0)
  compute(0, 0)
  fire_sc(0, 0)
  fire(2, 0)
  wait_gather(1, 1)
  compute(1, 1)
  fire_sc(1, 1)

  def pair(k, c):
    ci = 2 * k
    fire(ci + 1, 1)
    wait_gather(ci, 0)
    wait_sc(ci - 2, 0)
    compute(ci, 0)
    fire_sc(ci, 0)
    fire(ci + 2, 0)
    wait_gather(ci + 1, 1)
    wait_sc(ci - 1, 1)
    compute(ci + 1, 1)
    fire_sc(ci + 1, 1)
    return c

  lax.fori_loop(1, NCHUNK // 2 - 1, pair, 0)
  ci0 = NCHUNK - 2
  fire(ci0 + 1, 1)
  wait_gather(ci0, 0)
  wait_sc(ci0 - 2, 0)
  compute(ci0, 0)
  fire_sc(ci0, 0)
  wait_gather(ci0 + 1, 1)
  wait_sc(ci0 - 1, 1)
  compute(ci0 + 1, 1)
  fire_sc(ci0 + 1, 1)
  wait_sc(ci0, 0)
  wait_sc(ci0 + 1, 1)

  plsc.subcore_barrier()
  pltpu.sync_copy(s_sh.at[pl.ds(sid * RPT, RPT)],
                  s_out.at[cid, pl.ds(sid * RPT, RPT)])


_sc_passA = functools.partial(
    pl.kernel,
    out_type=jax.ShapeDtypeStruct((NC, NP, 16), jnp.float32),
    mesh=_MESH,
    compiler_params=pltpu.CompilerParams(use_tc_tiling_on_sc=False),
    scratch_types=[
        pltpu.VMEM((NCHUNK, B), jnp.int32),
        pltpu.VMEM((NCHUNK, B), jnp.int32),
        pltpu.VMEM((2, B, 16), jnp.float32),
        pltpu.VMEM((2, B, 16), jnp.float32),
        pltpu.VMEM((2, B, 16), jnp.float32),
        pltpu.VMEM((LANES,), jnp.float32),
        pltpu.VMEM_SHARED((NP, 16), jnp.float32),
        pltpu.SemaphoreType.DMA,
        pltpu.SemaphoreType.DMA,
        pltpu.SemaphoreType.DMA,
        pltpu.SemaphoreType.DMA,
    ],
)(_passA_body)


def _make_passB(d_row, n_heads):
  n_chunks = d_row // LANES
  splat_head = [(c if n_heads == HEADS else 0) for c in range(n_chunks)]
  ha_w = d_row + 16

  def body(esrc, edst, HA, avec, TS, out, src_all, dst_all, ha, ts, msg,
           a_v, o_sh, semA, semB, semSA, semSB):
    cid = lax.axis_index("c")
    sid = lax.axis_index("s")
    wid = cid * NS + sid
    has = [ha.at[0], ha.at[1]]
    tss = [ts.at[0], ts.at[1]]
    msgs = [msg.at[0], msg.at[1]]
    sems = [semA, semB]
    ssems = [semSA, semSB]

    def zero_row(i, c):
      for j in range(n_chunks):
        msg[0, i, pl.ds(j * LANES, LANES)] = jnp.zeros((LANES,), jnp.float32)
      return c

    lax.fori_loop(0, B, zero_row, 0)

    def zero_sh(k, c):
      pltpu.sync_copy(msgs[0], o_sh.at[pl.ds(sid * RPT + k * B, B)])
      return c

    lax.fori_loop(0, RPT // B, zero_sh, 0)
    plsc.subcore_barrier()

    pltpu.sync_copy(avec, a_v)
    av = a_v[:]
    pltpu.sync_copy(esrc.at[wid], src_all)
    pltpu.sync_copy(edst.at[wid], dst_all)

    def fire(ci, b):
      pltpu.async_copy(HA.at[src_all.at[ci]], has[b], sems[b])
      pltpu.async_copy(TS.at[dst_all.at[ci]], tss[b], sems[b])

    def wait_gather(ci, b):
      pltpu.make_async_copy(HA.at[src_all.at[ci]], has[b], sems[b]).wait()
      pltpu.make_async_copy(TS.at[dst_all.at[ci]], tss[b], sems[b]).wait()

    def compute(ci, b):
      def per_edge(i, cc):
        tsr = ha[b, i, pl.ds(d_row, LANES)]
        tdr = ts[b, i, :]
        w = _edge_w(tsr, tdr, av)
        attn = w * tdr
        for j in range(n_chunks):
          sp = jnp.take_along_axis(
              attn, jnp.full((LANES,), splat_head[j], jnp.int32), axis=0)
          msg[b, i, pl.ds(j * LANES, LANES)] = (
              ha[b, i, pl.ds(j * LANES, LANES)] * sp)
        return cc

      lax.fori_loop(0, B, per_edge, 0)

    def fire_sc(ci, b):
      pltpu.async_copy(msgs[b], o_sh.at[dst_all.at[ci]], ssems[b], add=True)

    def wait_sc(ci, b):
      pltpu.make_async_copy(
          msgs[b], o_sh.at[dst_all.at[ci]], ssems[b]).wait()

    fire(0, 0)
    fire(1, 1)
    wait_gather(0, 0)
    compute(0, 0)
    fire_sc(0, 0)
    fire(2, 0)
    wait_gather(1, 1)
    compute(1, 1)
    fire_sc(1, 1)

    def pair(k, c):
      ci = 2 * k
      fire(ci + 1, 1)
      wait_gather(ci, 0)
      wait_sc(ci - 2, 0)
      compute(ci, 0)
      fire_sc(ci, 0)
      fire(ci + 2, 0)
      wait_gather(ci + 1, 1)
      wait_sc(ci - 1, 1)
      compute(ci + 1, 1)
      fire_sc(ci + 1, 1)
      return c

    lax.fori_loop(1, NCHUNK // 2 - 1, pair, 0)
    ci0 = NCHUNK - 2
    fire(ci0 + 1, 1)
    wait_gather(ci0, 0)
    wait_sc(ci0 - 2, 0)
    compute(ci0, 0)
    fire_sc(ci0, 0)
    wait_gather(ci0 + 1, 1)
    wait_sc(ci0 - 1, 1)
    compute(ci0 + 1, 1)
    fire_sc(ci0 + 1, 1)
    wait_sc(ci0, 0)
    wait_sc(ci0 + 1, 1)

    plsc.subcore_barrier()
    pltpu.sync_copy(o_sh.at[pl.ds(sid * RPT, RPT)],
                    out.at[cid, pl.ds(sid * RPT, RPT)])

  return functools.partial(
      pl.kernel,
      out_type=jax.ShapeDtypeStruct((NC, NP, d_row), jnp.float32),
      mesh=_MESH,
      compiler_params=pltpu.CompilerParams(use_tc_tiling_on_sc=False),
      scratch_types=[
          pltpu.VMEM((NCHUNK, B), jnp.int32),
          pltpu.VMEM((NCHUNK, B), jnp.int32),
          pltpu.VMEM((2, B, ha_w), jnp.float32),
          pltpu.VMEM((2, B, 16), jnp.float32),
          pltpu.VMEM((2, B, d_row), jnp.float32),
          pltpu.VMEM((LANES,), jnp.float32),
          pltpu.VMEM_SHARED((NP, d_row), jnp.float32),
          pltpu.SemaphoreType.DMA,
          pltpu.SemaphoreType.DMA,
          pltpu.SemaphoreType.DMA,
          pltpu.SemaphoreType.DMA,
      ],
  )(body)


_sc_passB_128 = _make_passB(128, HEADS)


def _build_lr(al, ar, d, ph):
  rows = jnp.arange(d)
  hcol = rows // ph
  lr = jnp.zeros((d, 16), jnp.float32)
  lr = lr.at[rows, hcol].set(al.reshape(-1))
  lr = lr.at[rows, hcol + 8].set(ar.reshape(-1))
  return lr


def _layer(esrc, edst, p, bb_row, W, b_row, LR, d_in, d_out, first, passB):
  HA, T, A = _tc_prep(p, bb_row, W, b_row, LR, d_in, d_out, first)
  Av = A.reshape(16)
  sp = _sc_passA(esrc, edst, T, Av)
  TS = _tc_comb_ts(sp, T)
  return passB(esrc, edst, HA, Av, TS)


def kernel(x, edge_index, W0, b0, al0, ar0, bb0, W1, b1, al1, ar1, bb1,
           W2, b2, al2, ar2, bb2):
  ei = edge_index.astype(jnp.int32)
  esrc = ei[0].reshape(NW, NCHUNK, B)
  edst = ei[1].reshape(NW, NCHUNK, B)
  x_p = jnp.pad(x, ((0, NP - N), (0, 0)))

  LR0 = _build_lr(al0, ar0, 128, PH)
  LR1 = _build_lr(al1, ar1, 128, PH)
  rows2 = jnp.arange(NCLS)
  LR2 = jnp.zeros((128, 16), jnp.float32)
  for _j in range(8):
    LR2 = LR2.at[rows2, _j].set(al2.reshape(-1))
    LR2 = LR2.at[rows2, 8 + _j].set(ar2.reshape(-1))
  W2p = jnp.pad(W2, ((0, 0), (0, 128 - NCLS)))
  b2p = jnp.pad(b2, (0, 128 - NCLS))
  bb2p = jnp.pad(bb2, (0, 128 - NCLS))

  op0 = _layer(esrc, edst, x_p, b0.reshape(1, -1), W0, b0.reshape(1, -1),
               LR0, 128, 128, True, _sc_passB_128)
  op1 = _layer(esrc, edst, op0, bb0.reshape(1, -1), W1, b1.reshape(1, -1),
               LR1, 128, 128, False, _sc_passB_128)
  op2 = _layer(esrc, edst, op1, bb1.reshape(1, -1), W2p, b2p.reshape(1, -1),
               LR2, 128, 128, False, _sc_passB_128)

  outf = _tc_final(op2, bb2p.reshape(1, -1), 128)
  return outf[:N, :NCLS]


# speedup vs baseline: 80.0731x; 1.9444x over previous
import functools

import jax
import jax.numpy as jnp
from jax import lax
from jax.experimental import pallas as pl
from jax.experimental.pallas import tpu as pltpu
from jax.experimental.pallas import tpu_sc as plsc

N = 10000
E = 320000
D_IN = 128
HEADS = 8
PH = 16
HID = 128
NCLS = 40
SLOPE = 0.2

NC = 2
NS = 16
NW = NC * NS
LANES = 16

NP = 10240
BLK = 256
EW = E // NW
B = 40
NCHUNK = EW // B
RPT = NP // NS

_BIG = 1e30


def _leaky(v):
  return jnp.where(v >= 0, v, v * SLOPE)


def _prep_common(h_in, W_ref, b_ref, LR_ref, HA_ref, T_ref, A_ref, d_out):
  h = jnp.dot(h_in, W_ref[...], preferred_element_type=jnp.float32)
  h = h + b_ref[...]
  t = jnp.dot(h, LR_ref[...], preferred_element_type=jnp.float32)
  T_ref[...] = t
  HA_ref[:, 0:d_out] = h
  HA_ref[:, d_out:d_out + 8] = t[:, 0:8]
  HA_ref[:, d_out + 8:d_out + 16] = jnp.zeros((h.shape[0], 8), jnp.float32)
  blockmax = jnp.max(t[:, 0:8], axis=0, keepdims=True)
  cur = jnp.concatenate(
      [blockmax, jnp.full((1, 8), _BIG, jnp.float32)], axis=1)
  i = pl.program_id(0)

  @pl.when(i == 0)
  def _():
    A_ref[...] = cur

  @pl.when(i > 0)
  def _():
    A_ref[...] = jnp.maximum(A_ref[...], cur)


def _tc_prep(p, bb_row, W, b_row, LR, d_in, d_out, first):
  grid = NP // BLK

  def body(p_ref, bb_ref, W_ref, b_ref, LR_ref, HA_ref, T_ref, A_ref):
    if first:
      h_in = p_ref[...]
    else:
      s = p_ref[0] + p_ref[1] + bb_ref[...]
      h_in = jnp.where(s > 0, s, jnp.exp(s) - 1.0)
    _prep_common(h_in, W_ref, b_ref, LR_ref, HA_ref, T_ref, A_ref, d_out)

  in_spec_p = (
      pl.BlockSpec((BLK, d_in), lambda i: (i, 0)) if first
      else pl.BlockSpec((2, BLK, d_in), lambda i: (0, i, 0)))
  return pl.pallas_call(
      body,
      grid=(grid,),
      in_specs=[
          in_spec_p,
          pl.BlockSpec((1, d_in), lambda i: (0, 0)),
          pl.BlockSpec((d_in, d_out), lambda i: (0, 0)),
          pl.BlockSpec((1, d_out), lambda i: (0, 0)),
          pl.BlockSpec((d_out, 16), lambda i: (0, 0)),
      ],
      out_specs=[
          pl.BlockSpec((BLK, d_out + 16), lambda i: (i, 0)),
          pl.BlockSpec((BLK, 16), lambda i: (i, 0)),
          pl.BlockSpec((1, 16), lambda i: (0, 0)),
      ],
      out_shape=[
          jax.ShapeDtypeStruct((NP, d_out + 16), jnp.float32),
          jax.ShapeDtypeStruct((NP, 16), jnp.float32),
          jax.ShapeDtypeStruct((1, 16), jnp.float32),
      ],
  )(p, bb_row, W, b_row, LR)


def _comb_ts_body(sp_ref, t_ref, ts_ref):
  sinv = 1.0 / (sp_ref[0, :, 0:8] + sp_ref[1, :, 0:8] + 1e-16)
  ts_ref[...] = jnp.concatenate([sinv, t_ref[:, 8:16]], axis=1)


def _tc_comb_ts(sp, t):
  grid = NP // BLK
  return pl.pallas_call(
      _comb_ts_body,
      grid=(grid,),
      in_specs=[
          pl.BlockSpec((2, BLK, 16), lambda i: (0, i, 0)),
          pl.BlockSpec((BLK, 16), lambda i: (i, 0)),
      ],
      out_specs=pl.BlockSpec((BLK, 16), lambda i: (i, 0)),
      out_shape=jax.ShapeDtypeStruct((NP, 16), jnp.float32),
  )(sp, t)


def _final_body(p_ref, bb_ref, o_ref):
  o_ref[...] = p_ref[0] + p_ref[1] + bb_ref[...]


def _tc_final(p, bb_row, d_out):
  grid = NP // BLK
  return pl.pallas_call(
      _final_body,
      grid=(grid,),
      in_specs=[
          pl.BlockSpec((2, BLK, d_out), lambda i: (0, i, 0)),
          pl.BlockSpec((1, d_out), lambda i: (0, 0)),
      ],
      out_specs=pl.BlockSpec((BLK, d_out), lambda i: (i, 0)),
      out_shape=jax.ShapeDtypeStruct((NP, d_out), jnp.float32),
  )(p, bb_row)


_MESH = plsc.VectorSubcoreMesh(core_axis_name="c", subcore_axis_name="s")


def _edge_w(ts, td, av):
  rot_idx = (lax.iota(jnp.int32, LANES) & 7) + 8
  rot = jnp.take_along_axis(td, rot_idx, axis=0)
  e = _leaky(ts + rot)
  m = _leaky(rot + av)
  return jnp.exp(e - m)


def _passA_body(esrc, edst, T, avec, s_out, src_all, dst_all, tsrc, tdst,
                wbuf, a_v, s_sh, semA, semB, semSA, semSB):
  cid = lax.axis_index("c")
  sid = lax.axis_index("s")
  wid = cid * NS + sid
  tsrcs = [tsrc.at[0], tsrc.at[1]]
  tdsts = [tdst.at[0], tdst.at[1]]
  wbufs = [wbuf.at[0], wbuf.at[1]]
  sems = [semA, semB]
  ssems = [semSA, semSB]

  def zero_row(i, c):
    wbuf[0, i, :] = jnp.zeros((LANES,), jnp.float32)
    return c

  lax.fori_loop(0, B, zero_row, 0)

  def zero_sh(k, c):
    pltpu.sync_copy(wbufs[0], s_sh.at[pl.ds(sid * RPT + k * B, B)])
    return c

  lax.fori_loop(0, RPT // B, zero_sh, 0)
  plsc.subcore_barrier()

  pltpu.sync_copy(avec, a_v)
  av = a_v[:]
  pltpu.sync_copy(esrc.at[wid], src_all)
  pltpu.sync_copy(edst.at[wid], dst_all)

  def fire(ci, b):
    pltpu.async_copy(T.at[src_all.at[ci]], tsrcs[b], sems[b])
    pltpu.async_copy(T.at[dst_all.at[ci]], tdsts[b], sems[b])

  def wait_gather(ci, b):
    pltpu.make_async_copy(T.at[src_all.at[ci]], tsrcs[b], sems[b]).wait()
    pltpu.make_async_copy(T.at[dst_all.at[ci]], tdsts[b], sems[b]).wait()

  def compute(ci, b):
    @plsc.parallel_loop(0, B, step=1, unroll=4)
    def per_edge(i):
      wbuf[b, i, :] = _edge_w(tsrc[b, i, :], tdst[b, i, :], av)

  def fire_sc(ci, b):
    pltpu.async_copy(wbufs[b], s_sh.at[dst_all.at[ci]], ssems[b], add=True)

  def wait_sc(ci, b):
    pltpu.make_async_copy(wbufs[b], s_sh.at[dst_all.at[ci]], ssems[b]).wait()

  fire(0, 0)
  fire(1, 1)
  wait_gather(0, 0)
  compute(0, 0)
  fire_sc(0, 0)
  fire(2, 0)
  wait_gather(1, 1)
  compute(1, 1)
  fire_sc(1, 1)

  def pair(k, c):
    ci = 2 * k
    fire(ci + 1, 1)
    wait_gather(ci, 0)
    wait_sc(ci - 2, 0)
    compute(ci, 0)
    fire_sc(ci, 0)
    fire(ci + 2, 0)
    wait_gather(ci + 1, 1)
    wait_sc(ci - 1, 1)
    compute(ci + 1, 1)
    fire_sc(ci + 1, 1)
    return c

  lax.fori_loop(1, NCHUNK // 2 - 1, pair, 0)
  ci0 = NCHUNK - 2
  fire(ci0 + 1, 1)
  wait_gather(ci0, 0)
  wait_sc(ci0 - 2, 0)
  compute(ci0, 0)
  fire_sc(ci0, 0)
  wait_gather(ci0 + 1, 1)
  wait_sc(ci0 - 1, 1)
  compute(ci0 + 1, 1)
  fire_sc(ci0 + 1, 1)
  wait_sc(ci0, 0)
  wait_sc(ci0 + 1, 1)

  plsc.subcore_barrier()
  pltpu.sync_copy(s_sh.at[pl.ds(sid * RPT, RPT)],
                  s_out.at[cid, pl.ds(sid * RPT, RPT)])


_sc_passA = functools.partial(
    pl.kernel,
    out_type=jax.ShapeDtypeStruct((NC, NP, 16), jnp.float32),
    mesh=_MESH,
    compiler_params=pltpu.CompilerParams(use_tc_tiling_on_sc=False),
    scratch_types=[
        pltpu.VMEM((NCHUNK, B), jnp.int32),
        pltpu.VMEM((NCHUNK, B), jnp.int32),
        pltpu.VMEM((2, B, 16), jnp.float32),
        pltpu.VMEM((2, B, 16), jnp.float32),
        pltpu.VMEM((2, B, 16), jnp.float32),
        pltpu.VMEM((LANES,), jnp.float32),
        pltpu.VMEM_SHARED((NP, 16), jnp.float32),
        pltpu.SemaphoreType.DMA,
        pltpu.SemaphoreType.DMA,
        pltpu.SemaphoreType.DMA,
        pltpu.SemaphoreType.DMA,
    ],
)(_passA_body)


def _make_passB(d_row, n_heads):
  n_chunks = d_row // LANES
  splat_head = [(c if n_heads == HEADS else 0) for c in range(n_chunks)]
  ha_w = d_row + 16

  def body(esrc, edst, HA, avec, TS, out, src_all, dst_all, ha, ts, msg,
           a_v, o_sh, semA, semB, semSA, semSB):
    cid = lax.axis_index("c")
    sid = lax.axis_index("s")
    wid = cid * NS + sid
    has = [ha.at[0], ha.at[1]]
    tss = [ts.at[0], ts.at[1]]
    msgs = [msg.at[0], msg.at[1]]
    sems = [semA, semB]
    ssems = [semSA, semSB]

    def zero_row(i, c):
      for j in range(n_chunks):
        msg[0, i, pl.ds(j * LANES, LANES)] = jnp.zeros((LANES,), jnp.float32)
      return c

    lax.fori_loop(0, B, zero_row, 0)

    def zero_sh(k, c):
      pltpu.sync_copy(msgs[0], o_sh.at[pl.ds(sid * RPT + k * B, B)])
      return c

    lax.fori_loop(0, RPT // B, zero_sh, 0)
    plsc.subcore_barrier()

    pltpu.sync_copy(avec, a_v)
    av = a_v[:]
    pltpu.sync_copy(esrc.at[wid], src_all)
    pltpu.sync_copy(edst.at[wid], dst_all)

    def fire(ci, b):
      pltpu.async_copy(HA.at[src_all.at[ci]], has[b], sems[b])
      pltpu.async_copy(TS.at[dst_all.at[ci]], tss[b], sems[b])

    def wait_gather(ci, b):
      pltpu.make_async_copy(HA.at[src_all.at[ci]], has[b], sems[b]).wait()
      pltpu.make_async_copy(TS.at[dst_all.at[ci]], tss[b], sems[b]).wait()

    def compute(ci, b):
      @plsc.parallel_loop(0, B, step=1, unroll=4)
      def per_edge(i):
        tsr = ha[b, i, pl.ds(d_row, LANES)]
        tdr = ts[b, i, :]
        w = _edge_w(tsr, tdr, av)
        attn = w * tdr
        for j in range(n_chunks):
          sp = jnp.take_along_axis(
              attn, jnp.full((LANES,), splat_head[j], jnp.int32), axis=0)
          msg[b, i, pl.ds(j * LANES, LANES)] = (
              ha[b, i, pl.ds(j * LANES, LANES)] * sp)

    def fire_sc(ci, b):
      pltpu.async_copy(msgs[b], o_sh.at[dst_all.at[ci]], ssems[b], add=True)

    def wait_sc(ci, b):
      pltpu.make_async_copy(
          msgs[b], o_sh.at[dst_all.at[ci]], ssems[b]).wait()

    fire(0, 0)
    fire(1, 1)
    wait_gather(0, 0)
    compute(0, 0)
    fire_sc(0, 0)
    fire(2, 0)
    wait_gather(1, 1)
    compute(1, 1)
    fire_sc(1, 1)

    def pair(k, c):
      ci = 2 * k
      fire(ci + 1, 1)
      wait_gather(ci, 0)
      wait_sc(ci - 2, 0)
      compute(ci, 0)
      fire_sc(ci, 0)
      fire(ci + 2, 0)
      wait_gather(ci + 1, 1)
      wait_sc(ci - 1, 1)
      compute(ci + 1, 1)
      fire_sc(ci + 1, 1)
      return c

    lax.fori_loop(1, NCHUNK // 2 - 1, pair, 0)
    ci0 = NCHUNK - 2
    fire(ci0 + 1, 1)
    wait_gather(ci0, 0)
    wait_sc(ci0 - 2, 0)
    compute(ci0, 0)
    fire_sc(ci0, 0)
    wait_gather(ci0 + 1, 1)
    wait_sc(ci0 - 1, 1)
    compute(ci0 + 1, 1)
    fire_sc(ci0 + 1, 1)
    wait_sc(ci0, 0)
    wait_sc(ci0 + 1, 1)

    plsc.subcore_barrier()
    pltpu.sync_copy(o_sh.at[pl.ds(sid * RPT, RPT)],
                    out.at[cid, pl.ds(sid * RPT, RPT)])

  return functools.partial(
      pl.kernel,
      out_type=jax.ShapeDtypeStruct((NC, NP, d_row), jnp.float32),
      mesh=_MESH,
      compiler_params=pltpu.CompilerParams(use_tc_tiling_on_sc=False),
      scratch_types=[
          pltpu.VMEM((NCHUNK, B), jnp.int32),
          pltpu.VMEM((NCHUNK, B), jnp.int32),
          pltpu.VMEM((2, B, ha_w), jnp.float32),
          pltpu.VMEM((2, B, 16), jnp.float32),
          pltpu.VMEM((2, B, d_row), jnp.float32),
          pltpu.VMEM((LANES,), jnp.float32),
          pltpu.VMEM_SHARED((NP, d_row), jnp.float32),
          pltpu.SemaphoreType.DMA,
          pltpu.SemaphoreType.DMA,
          pltpu.SemaphoreType.DMA,
          pltpu.SemaphoreType.DMA,
      ],
  )(body)


_sc_passB_128 = _make_passB(128, HEADS)


def _build_lr(al, ar, d, ph):
  rows = jnp.arange(d)
  hcol = rows // ph
  lr = jnp.zeros((d, 16), jnp.float32)
  lr = lr.at[rows, hcol].set(al.reshape(-1))
  lr = lr.at[rows, hcol + 8].set(ar.reshape(-1))
  return lr


def _layer(esrc, edst, p, bb_row, W, b_row, LR, d_in, d_out, first, passB):
  HA, T, A = _tc_prep(p, bb_row, W, b_row, LR, d_in, d_out, first)
  Av = A.reshape(16)
  sp = _sc_passA(esrc, edst, T, Av)
  TS = _tc_comb_ts(sp, T)
  return passB(esrc, edst, HA, Av, TS)


def kernel(x, edge_index, W0, b0, al0, ar0, bb0, W1, b1, al1, ar1, bb1,
           W2, b2, al2, ar2, bb2):
  ei = edge_index.astype(jnp.int32)
  esrc = ei[0].reshape(NW, NCHUNK, B)
  edst = ei[1].reshape(NW, NCHUNK, B)
  x_p = jnp.pad(x, ((0, NP - N), (0, 0)))

  LR0 = _build_lr(al0, ar0, 128, PH)
  LR1 = _build_lr(al1, ar1, 128, PH)
  rows2 = jnp.arange(NCLS)
  LR2 = jnp.zeros((128, 16), jnp.float32)
  for _j in range(8):
    LR2 = LR2.at[rows2, _j].set(al2.reshape(-1))
    LR2 = LR2.at[rows2, 8 + _j].set(ar2.reshape(-1))
  W2p = jnp.pad(W2, ((0, 0), (0, 128 - NCLS)))
  b2p = jnp.pad(b2, (0, 128 - NCLS))
  bb2p = jnp.pad(bb2, (0, 128 - NCLS))

  op0 = _layer(esrc, edst, x_p, b0.reshape(1, -1), W0, b0.reshape(1, -1),
               LR0, 128, 128, True, _sc_passB_128)
  op1 = _layer(esrc, edst, op0, bb0.reshape(1, -1), W1, b1.reshape(1, -1),
               LR1, 128, 128, False, _sc_passB_128)
  op2 = _layer(esrc, edst, op1, bb1.reshape(1, -1), W2p, b2p.reshape(1, -1),
               LR2, 128, 128, False, _sc_passB_128)

  outf = _tc_final(op2, bb2p.reshape(1, -1), 128)
  return outf[:N, :NCLS]


# speedup vs baseline: 82.8700x; 1.0349x over previous
import functools

import jax
import jax.numpy as jnp
from jax import lax
from jax.experimental import pallas as pl
from jax.experimental.pallas import tpu as pltpu
from jax.experimental.pallas import tpu_sc as plsc

N = 10000
E = 320000
D_IN = 128
HEADS = 8
PH = 16
HID = 128
NCLS = 40
SLOPE = 0.2

NC = 2
NS = 16
NW = NC * NS
LANES = 16

NP = 10240
BLK = 256
EW = E // NW
B = 40
NCHUNK = EW // B
RPT = NP // NS

_BIG = 1e30


def _leaky(v):
  return jnp.where(v >= 0, v, v * SLOPE)


def _prep_common(h_in, W_ref, b_ref, LR_ref, H_ref, T_ref, A_ref, d_out):
  h = jnp.dot(h_in, W_ref[...], preferred_element_type=jnp.float32)
  h = h + b_ref[...]
  t = jnp.dot(h, LR_ref[...], preferred_element_type=jnp.float32)
  T_ref[...] = t
  H_ref[...] = h.astype(jnp.bfloat16)
  blockmax = jnp.max(t[:, 0:8], axis=0, keepdims=True)
  cur = jnp.concatenate(
      [blockmax, jnp.full((1, 8), _BIG, jnp.float32)], axis=1)
  i = pl.program_id(0)

  @pl.when(i == 0)
  def _():
    A_ref[...] = cur

  @pl.when(i > 0)
  def _():
    A_ref[...] = jnp.maximum(A_ref[...], cur)


def _tc_prep(p, bb_row, W, b_row, LR, d_in, d_out, first):
  grid = NP // BLK

  def body(p_ref, bb_ref, W_ref, b_ref, LR_ref, H_ref, T_ref, A_ref):
    if first:
      h_in = p_ref[...]
    else:
      s = p_ref[0] + p_ref[1] + bb_ref[...]
      h_in = jnp.where(s > 0, s, jnp.exp(s) - 1.0)
    _prep_common(h_in, W_ref, b_ref, LR_ref, H_ref, T_ref, A_ref, d_out)

  in_spec_p = (
      pl.BlockSpec((BLK, d_in), lambda i: (i, 0)) if first
      else pl.BlockSpec((2, BLK, d_in), lambda i: (0, i, 0)))
  return pl.pallas_call(
      body,
      grid=(grid,),
      in_specs=[
          in_spec_p,
          pl.BlockSpec((1, d_in), lambda i: (0, 0)),
          pl.BlockSpec((d_in, d_out), lambda i: (0, 0)),
          pl.BlockSpec((1, d_out), lambda i: (0, 0)),
          pl.BlockSpec((d_out, 16), lambda i: (0, 0)),
      ],
      out_specs=[
          pl.BlockSpec((BLK, d_out), lambda i: (i, 0)),
          pl.BlockSpec((BLK, 16), lambda i: (i, 0)),
          pl.BlockSpec((1, 16), lambda i: (0, 0)),
      ],
      out_shape=[
          jax.ShapeDtypeStruct((NP, d_out), jnp.bfloat16),
          jax.ShapeDtypeStruct((NP, 16), jnp.float32),
          jax.ShapeDtypeStruct((1, 16), jnp.float32),
      ],
  )(p, bb_row, W, b_row, LR)


def _final_body(p_ref, bb_ref, o_ref):
  o_ref[...] = p_ref[0] + p_ref[1] + bb_ref[...]


def _tc_final(p, bb_row, d_out):
  grid = NP // BLK
  return pl.pallas_call(
      _final_body,
      grid=(grid,),
      in_specs=[
          pl.BlockSpec((2, BLK, d_out), lambda i: (0, i, 0)),
          pl.BlockSpec((1, d_out), lambda i: (0, 0)),
      ],
      out_specs=pl.BlockSpec((BLK, d_out), lambda i: (i, 0)),
      out_shape=jax.ShapeDtypeStruct((NP, d_out), jnp.float32),
  )(p, bb_row)


_MESH = plsc.VectorSubcoreMesh(core_axis_name="c", subcore_axis_name="s")


def _edge_w(ts, td, av):
  rot_idx = (lax.iota(jnp.int32, LANES) & 7) + 8
  rot = jnp.take_along_axis(td, rot_idx, axis=0)
  e = _leaky(ts + rot)
  m = _leaky(rot + av)
  return jnp.exp(e - m)


def _passA_body(esrc, edst, T, avec, s_out, src_all, dst_all, tsrc, tdst,
                wbuf, a_v, s_sh, semA, semB, semSA, semSB):
  cid = lax.axis_index("c")
  sid = lax.axis_index("s")
  wid = cid * NS + sid
  tsrcs = [tsrc.at[0], tsrc.at[1]]
  tdsts = [tdst.at[0], tdst.at[1]]
  wbufs = [wbuf.at[0], wbuf.at[1]]
  sems = [semA, semB]
  ssems = [semSA, semSB]

  def zero_row(i, c):
    wbuf[0, i, :] = jnp.zeros((LANES,), jnp.float32)
    return c

  lax.fori_loop(0, B, zero_row, 0)

  def zero_sh(k, c):
    pltpu.sync_copy(wbufs[0], s_sh.at[pl.ds(sid * RPT + k * B, B)])
    return c

  lax.fori_loop(0, RPT // B, zero_sh, 0)
  plsc.subcore_barrier()

  pltpu.sync_copy(avec, a_v)
  av = a_v[:]
  pltpu.sync_copy(esrc.at[wid], src_all)
  pltpu.sync_copy(edst.at[wid], dst_all)

  def fire(ci, b):
    pltpu.async_copy(T.at[src_all.at[ci]], tsrcs[b], sems[b])
    pltpu.async_copy(T.at[dst_all.at[ci]], tdsts[b], sems[b])

  def wait_gather(ci, b):
    pltpu.make_async_copy(T.at[src_all.at[ci]], tsrcs[b], sems[b]).wait()
    pltpu.make_async_copy(T.at[dst_all.at[ci]], tdsts[b], sems[b]).wait()

  def compute(ci, b):
    @plsc.parallel_loop(0, B, step=1, unroll=4)
    def per_edge(i):
      wbuf[b, i, :] = _edge_w(tsrc[b, i, :], tdst[b, i, :], av)

  def fire_sc(ci, b):
    pltpu.async_copy(wbufs[b], s_sh.at[dst_all.at[ci]], ssems[b], add=True)

  def wait_sc(ci, b):
    pltpu.make_async_copy(wbufs[b], s_sh.at[dst_all.at[ci]], ssems[b]).wait()

  fire(0, 0)
  fire(1, 1)
  wait_gather(0, 0)
  compute(0, 0)
  fire_sc(0, 0)
  fire(2, 0)
  wait_gather(1, 1)
  compute(1, 1)
  fire_sc(1, 1)

  def pair(k, c):
    ci = 2 * k
    fire(ci + 1, 1)
    wait_gather(ci, 0)
    wait_sc(ci - 2, 0)
    compute(ci, 0)
    fire_sc(ci, 0)
    fire(ci + 2, 0)
    wait_gather(ci + 1, 1)
    wait_sc(ci - 1, 1)
    compute(ci + 1, 1)
    fire_sc(ci + 1, 1)
    return c

  lax.fori_loop(1, NCHUNK // 2 - 1, pair, 0)
  ci0 = NCHUNK - 2
  fire(ci0 + 1, 1)
  wait_gather(ci0, 0)
  wait_sc(ci0 - 2, 0)
  compute(ci0, 0)
  fire_sc(ci0, 0)
  wait_gather(ci0 + 1, 1)
  wait_sc(ci0 - 1, 1)
  compute(ci0 + 1, 1)
  fire_sc(ci0 + 1, 1)
  wait_sc(ci0, 0)
  wait_sc(ci0 + 1, 1)

  plsc.subcore_barrier()
  pltpu.sync_copy(s_sh.at[pl.ds(sid * RPT, RPT)],
                  s_out.at[cid, pl.ds(sid * RPT, RPT)])


_sc_passA = functools.partial(
    pl.kernel,
    out_type=jax.ShapeDtypeStruct((NC, NP, 16), jnp.float32),
    mesh=_MESH,
    compiler_params=pltpu.CompilerParams(
        use_tc_tiling_on_sc=False, needs_layout_passes=False),
    scratch_types=[
        pltpu.VMEM((NCHUNK, B), jnp.int32),
        pltpu.VMEM((NCHUNK, B), jnp.int32),
        pltpu.VMEM((2, B, 16), jnp.float32),
        pltpu.VMEM((2, B, 16), jnp.float32),
        pltpu.VMEM((2, B, 16), jnp.float32),
        pltpu.VMEM((LANES,), jnp.float32),
        pltpu.VMEM_SHARED((NP, 16), jnp.float32),
        pltpu.SemaphoreType.DMA,
        pltpu.SemaphoreType.DMA,
        pltpu.SemaphoreType.DMA,
        pltpu.SemaphoreType.DMA,
    ],
)(_passA_body)


def _passB_body(esrc, edst, HP, avec, T, SP0, SP1, out, src_all, dst_all,
                hp, tr, s0r, s1r, msg, a_v, o_sh, semA, semB, semSA, semSB):
  cid = lax.axis_index("c")
  sid = lax.axis_index("s")
  wid = cid * NS + sid
  hps = [hp.at[0], hp.at[1]]
  trs = [tr.at[0], tr.at[1]]
  s0s = [s0r.at[0], s0r.at[1]]
  s1s = [s1r.at[0], s1r.at[1]]
  msgs = [msg.at[0], msg.at[1]]
  sems = [semA, semB]
  ssems = [semSA, semSB]

  def zero_row(i, c):
    for j in range(8):
      msg[0, i, pl.ds(j * LANES, LANES)] = jnp.zeros((LANES,), jnp.float32)
    return c

  lax.fori_loop(0, B, zero_row, 0)

  def zero_sh(k, c):
    pltpu.sync_copy(msgs[0], o_sh.at[pl.ds(sid * RPT + k * B, B)])
    return c

  lax.fori_loop(0, RPT // B, zero_sh, 0)
  plsc.subcore_barrier()

  pltpu.sync_copy(avec, a_v)
  av = a_v[:]
  pltpu.sync_copy(esrc.at[wid], src_all)
  pltpu.sync_copy(edst.at[wid], dst_all)

  def fire(ci, b):
    pltpu.async_copy(HP.at[src_all.at[ci]], hps[b], sems[b])
    pltpu.async_copy(T.at[dst_all.at[ci]], trs[b], sems[b])
    pltpu.async_copy(SP0.at[dst_all.at[ci]], s0s[b], sems[b])
    pltpu.async_copy(SP1.at[dst_all.at[ci]], s1s[b], sems[b])

  def wait_gather(ci, b):
    pltpu.make_async_copy(HP.at[src_all.at[ci]], hps[b], sems[b]).wait()
    pltpu.make_async_copy(T.at[dst_all.at[ci]], trs[b], sems[b]).wait()
    pltpu.make_async_copy(SP0.at[dst_all.at[ci]], s0s[b], sems[b]).wait()
    pltpu.make_async_copy(SP1.at[dst_all.at[ci]], s1s[b], sems[b]).wait()

  def compute(ci, b):
    @plsc.parallel_loop(0, B, step=1, unroll=4)
    def per_edge(i):
      tsr = hp[b, i, pl.ds(64, LANES)]
      tdr = tr[b, i, :]
      w = _edge_w(tsr, tdr, av)
      srow = s0r[b, i, :] + s1r[b, i, :]
      attn = w / (srow + 1e-16)
      for jj in range(4):
        p = hp[b, i, pl.ds(16 * jj, LANES)]
        pb = plsc.bitcast(p, jnp.bfloat16)
        e0, e1 = plsc.unpack(
            pb, format=plsc.PackFormat.INTERLEAVED,
            preferred_element_type=jnp.float32)
        sp0 = jnp.take_along_axis(
            attn, jnp.full((LANES,), 2 * jj, jnp.int32), axis=0)
        sp1 = jnp.take_along_axis(
            attn, jnp.full((LANES,), 2 * jj + 1, jnp.int32), axis=0)
        msg[b, i, pl.ds(32 * jj, LANES)] = e0 * sp0
        msg[b, i, pl.ds(32 * jj + 16, LANES)] = e1 * sp1

  def fire_sc(ci, b):
    pltpu.async_copy(msgs[b], o_sh.at[dst_all.at[ci]], ssems[b], add=True)

  def wait_sc(ci, b):
    pltpu.make_async_copy(msgs[b], o_sh.at[dst_all.at[ci]], ssems[b]).wait()

  fire(0, 0)
  fire(1, 1)
  wait_gather(0, 0)
  compute(0, 0)
  fire_sc(0, 0)
  fire(2, 0)
  wait_gather(1, 1)
  compute(1, 1)
  fire_sc(1, 1)

  def pair(k, c):
    ci = 2 * k
    fire(ci + 1, 1)
    wait_gather(ci, 0)
    wait_sc(ci - 2, 0)
    compute(ci, 0)
    fire_sc(ci, 0)
    fire(ci + 2, 0)
    wait_gather(ci + 1, 1)
    wait_sc(ci - 1, 1)
    compute(ci + 1, 1)
    fire_sc(ci + 1, 1)
    return c

  lax.fori_loop(1, NCHUNK // 2 - 1, pair, 0)
  ci0 = NCHUNK - 2
  fire(ci0 + 1, 1)
  wait_gather(ci0, 0)
  wait_sc(ci0 - 2, 0)
  compute(ci0, 0)
  fire_sc(ci0, 0)
  wait_gather(ci0 + 1, 1)
  wait_sc(ci0 - 1, 1)
  compute(ci0 + 1, 1)
  fire_sc(ci0 + 1, 1)
  wait_sc(ci0, 0)
  wait_sc(ci0 + 1, 1)

  plsc.subcore_barrier()
  pltpu.sync_copy(o_sh.at[pl.ds(sid * RPT, RPT)],
                  out.at[cid, pl.ds(sid * RPT, RPT)])


_sc_passB = functools.partial(
    pl.kernel,
    out_type=jax.ShapeDtypeStruct((NC, NP, 128), jnp.float32),
    mesh=_MESH,
    compiler_params=pltpu.CompilerParams(
        use_tc_tiling_on_sc=False, needs_layout_passes=False),
    scratch_types=[
        pltpu.VMEM((NCHUNK, B), jnp.int32),
        pltpu.VMEM((NCHUNK, B), jnp.int32),
        pltpu.VMEM((2, B, 80), jnp.float32),
        pltpu.VMEM((2, B, 16), jnp.float32),
        pltpu.VMEM((2, B, 16), jnp.float32),
        pltpu.VMEM((2, B, 16), jnp.float32),
        pltpu.VMEM((2, B, 128), jnp.float32),
        pltpu.VMEM((LANES,), jnp.float32),
        pltpu.VMEM_SHARED((NP, 128), jnp.float32),
        pltpu.SemaphoreType.DMA,
        pltpu.SemaphoreType.DMA,
        pltpu.SemaphoreType.DMA,
        pltpu.SemaphoreType.DMA,
    ],
)(_passB_body)


def _build_lr(al, ar, d, ph):
  rows = jnp.arange(d)
  hcol = rows // ph
  lr = jnp.zeros((d, 16), jnp.float32)
  lr = lr.at[rows, hcol].set(al.reshape(-1))
  lr = lr.at[rows, hcol + 8].set(ar.reshape(-1))
  return lr


def _layer(esrc, edst, p, bb_row, W, b_row, LR, d_in, d_out, first):
  Hbf, T, A = _tc_prep(p, bb_row, W, b_row, LR, d_in, d_out, first)
  t4 = Hbf.reshape(NP, 4, 2, 16).transpose(0, 1, 3, 2)
  packed = jax.lax.bitcast_convert_type(t4, jnp.float32).reshape(NP, 64)
  HP = jnp.concatenate(
      [packed, T[:, 0:8], jnp.zeros((NP, 8), jnp.float32)], axis=1)
  Av = A.reshape(16)
  sp = _sc_passA(esrc, edst, T, Av)
  return _sc_passB(esrc, edst, HP, Av, T, sp[0], sp[1])


def kernel(x, edge_index, W0, b0, al0, ar0, bb0, W1, b1, al1, ar1, bb1,
           W2, b2, al2, ar2, bb2):
  ei = edge_index.astype(jnp.int32)
  esrc = ei[0].reshape(NW, NCHUNK, B)
  edst = ei[1].reshape(NW, NCHUNK, B)
  x_p = jnp.pad(x, ((0, NP - N), (0, 0)))

  LR0 = _build_lr(al0, ar0, 128, PH)
  LR1 = _build_lr(al1, ar1, 128, PH)
  rows2 = jnp.arange(NCLS)
  LR2 = jnp.zeros((128, 16), jnp.float32)
  for _j in range(8):
    LR2 = LR2.at[rows2, _j].set(al2.reshape(-1))
    LR2 = LR2.at[rows2, 8 + _j].set(ar2.reshape(-1))
  W2p = jnp.pad(W2, ((0, 0), (0, 128 - NCLS)))
  b2p = jnp.pad(b2, (0, 128 - NCLS))
  bb2p = jnp.pad(bb2, (0, 128 - NCLS))

  op0 = _layer(esrc, edst, x_p, b0.reshape(1, -1), W0, b0.reshape(1, -1),
               LR0, 128, 128, True)
  op1 = _layer(esrc, edst, op0, bb0.reshape(1, -1), W1, b1.reshape(1, -1),
               LR1, 128, 128, False)
  op2 = _layer(esrc, edst, op1, bb1.reshape(1, -1), W2p, b2p.reshape(1, -1),
               LR2, 128, 128, False)

  outf = _tc_final(op2, bb2p.reshape(1, -1), 128)
  return outf[:N, :NCLS]
